# baseline (speedup 1.0000x reference)
import functools

import jax
import jax.numpy as jnp
from jax import lax
from jax.experimental import pallas as pl
from jax.experimental.pallas import tpu as pltpu
from jax.experimental.pallas import tpu_sc as plsc

VOCAB = 100000
E = 128
B = 4096
S = 20
LANES = 16

NC, NS = 2, 16
NW = NC * NS
BPW = B // NW
GN = 80
IDX_ROWS = BPW * S // GN
TPC = 320
CHUNKS = [(i * 16, 16) for i in range(7)] + [(112, 8), (120, 8)]

_mesh = plsc.VectorSubcoreMesh(core_axis_name="c", subcore_axis_name="s")


@functools.partial(
    pl.kernel,
    out_type=jax.ShapeDtypeStruct((B, E), jnp.float32),
    mesh=_mesh,
    scratch_types=[
        pltpu.VMEM((IDX_ROWS, GN), jnp.int32),
        pltpu.VMEM((TPC, E), jnp.float32),
        pltpu.VMEM((TPC, E), jnp.float32),
        pltpu.VMEM((16, E), jnp.float32),
        pltpu.VMEM((16, E), jnp.float32),
        pltpu.SemaphoreType.DMA,
        pltpu.SemaphoreType.DMA,
        pltpu.SemaphoreType.DMA,
    ],
)
def _sc_embed_sum(utt_hbm, w_hbm, out_hbm, idx_v, rows0, rows1, outv0,
                  outv1, sem0, sem1, sem_out):
    wid = lax.axis_index("s") * NC + lax.axis_index("c")
    pltpu.sync_copy(utt_hbm.at[wid, pl.ds(0, 8)], idx_v.at[pl.ds(0, 8)])

    bufs = (rows0, rows1)
    sems = (sem0, sem1)
    outs = (outv0, outv1)

    def fire(c):
        base, nb = CHUNKS[c]
        buf = bufs[c % 2]
        sem = sems[c % 2]
        row0 = base * S // GN
        for j in range(nb * S // GN):
            pltpu.async_copy(
                w_hbm.at[idx_v.at[row0 + j]],
                buf.at[pl.ds(j * GN, GN)],
                sem,
            )

    fire(0)
    pltpu.sync_copy(utt_hbm.at[wid, pl.ds(8, IDX_ROWS - 8)],
                    idx_v.at[pl.ds(8, IDX_ROWS - 8)])
    store_handles = [None, None]
    for c in range(len(CHUNKS)):
        if c + 1 < len(CHUNKS):
            fire(c + 1)
        base, nb = CHUNKS[c]
        buf = bufs[c % 2]
        pltpu.make_async_copy(w_hbm.at[pl.ds(0, nb * S)],
                              buf.at[pl.ds(0, nb * S)], sems[c % 2]).wait()
        out_v = outs[c % 2]
        if store_handles[c % 2] is not None:
            store_handles[c % 2].wait()

        def accum(b, _, buf=buf, out_v=out_v):
            cols = [pl.ds(eb * LANES, LANES) for eb in range(E // LANES)]
            accs = [buf[b * S, col] for col in cols]
            for s in range(1, S):
                row = b * S + s
                accs = [acc + buf[row, col] for acc, col in zip(accs, cols)]
            for col, acc in zip(cols, accs):
                out_v[b, col] = acc
            return 0

        lax.fori_loop(0, nb, accum, 0)
        store_handles[c % 2] = pltpu.async_copy(
            out_v.at[pl.ds(0, nb)],
            out_hbm.at[pl.ds(wid * BPW + base, nb)], sem_out)
    for h in store_handles:
        h.wait()


def kernel(utterance, W):
    utt = utterance.astype(jnp.int32).reshape(NW, IDX_ROWS, GN)
    return _sc_embed_sum(utt, W)


# speedup vs baseline: 1.0050x; 1.0050x over previous
import functools

import jax
import jax.numpy as jnp
from jax import lax
from jax.experimental import pallas as pl
from jax.experimental.pallas import tpu as pltpu
from jax.experimental.pallas import tpu_sc as plsc

VOCAB = 100000
E = 128
B = 4096
S = 20
LANES = 16

NC, NS = 2, 16
NW = NC * NS
BPW = B // NW
CB = 16
NCHUNK = BPW // CB
TPC = CB * S
GN = 64
NG = TPC // GN
IDX_ROWS = BPW * S // GN

_mesh = plsc.VectorSubcoreMesh(core_axis_name="c", subcore_axis_name="s")


@functools.partial(
    pl.kernel,
    out_type=jax.ShapeDtypeStruct((B, E), jnp.float32),
    mesh=_mesh,
    scratch_types=[
        pltpu.VMEM((IDX_ROWS, GN), jnp.int32),
        pltpu.VMEM((TPC, E), jnp.float32),
        pltpu.VMEM((TPC, E), jnp.float32),
        pltpu.VMEM((CB, E), jnp.float32),
        pltpu.VMEM((CB, E), jnp.float32),
        pltpu.SemaphoreType.DMA,
        pltpu.SemaphoreType.DMA,
        pltpu.SemaphoreType.DMA,
    ],
)
def _sc_embed_sum(utt_hbm, w_hbm, out_hbm, idx_v, rows0, rows1, outv0,
                  outv1, sem0, sem1, sem_out):
    wid = lax.axis_index("s") * NC + lax.axis_index("c")
    pltpu.sync_copy(utt_hbm.at[wid], idx_v)

    bufs = (rows0, rows1)
    sems = (sem0, sem1)
    outs = (outv0, outv1)

    def fire(c):
        buf = bufs[c % 2]
        sem = sems[c % 2]
        return [
            pltpu.async_copy(
                w_hbm.at[idx_v.at[c * NG + j]],
                buf.at[pl.ds(j * GN, GN)],
                sem,
            )
            for j in range(NG)
        ]

    fire(0)
    store_handles = [None, None]
    for c in range(NCHUNK):
        if c + 1 < NCHUNK:
            fire(c + 1)
        buf = bufs[c % 2]
        pltpu.make_async_copy(w_hbm.at[pl.ds(0, TPC)], buf,
                              sems[c % 2]).wait()
        out_v = outs[c % 2]
        if store_handles[c % 2] is not None:
            store_handles[c % 2].wait()

        def accum(b, _, buf=buf, out_v=out_v):
            cols = [pl.ds(eb * LANES, LANES) for eb in range(E // LANES)]
            accs = [buf[b * S, col] for col in cols]
            for s in range(1, S):
                row = b * S + s
                accs = [acc + buf[row, col] for acc, col in zip(accs, cols)]
            for col, acc in zip(cols, accs):
                out_v[b, col] = acc
            return 0

        lax.fori_loop(0, CB, accum, 0)
        store_handles[c % 2] = pltpu.async_copy(
            out_v, out_hbm.at[pl.ds(wid * BPW + c * CB, CB)], sem_out)
    for h in store_handles:
        h.wait()


def kernel(utterance, W):
    utt = utterance.astype(jnp.int32).reshape(NW, IDX_ROWS, GN)
    return _sc_embed_sum(utt, W)


# speedup vs baseline: 1.0091x; 1.0041x over previous
import functools

import jax
import jax.numpy as jnp
from jax import lax
from jax.experimental import pallas as pl
from jax.experimental.pallas import tpu as pltpu
from jax.experimental.pallas import tpu_sc as plsc

VOCAB = 100000
E = 128
B = 4096
S = 20
LANES = 16

NC, NS = 2, 16
NW = NC * NS
BPW = B // NW
CB = 16
NCHUNK = BPW // CB
TPC = CB * S
GN = 64
NG = TPC // GN
IDX_ROWS = BPW * S // GN

_mesh = plsc.VectorSubcoreMesh(core_axis_name="c", subcore_axis_name="s")


@functools.partial(
    pl.kernel,
    out_type=jax.ShapeDtypeStruct((B, E), jnp.float32),
    mesh=_mesh,
    scratch_types=[
        pltpu.VMEM((IDX_ROWS, GN), jnp.int32),
        pltpu.VMEM((TPC, E), jnp.float32),
        pltpu.VMEM((TPC, E), jnp.float32),
        pltpu.VMEM((CB, E), jnp.float32),
        pltpu.VMEM((CB, E), jnp.float32),
        pltpu.SemaphoreType.DMA,
        pltpu.SemaphoreType.DMA,
        pltpu.SemaphoreType.DMA,
    ],
)
def _sc_embed_sum(utt_hbm, w_hbm, out_hbm, idx_v, rows0, rows1, outv0,
                  outv1, sem0, sem1, sem_out):
    wid = lax.axis_index("s") * NC + lax.axis_index("c")
    pltpu.sync_copy(utt_hbm.at[wid], idx_v)

    bufs = (rows0, rows1)
    sems = (sem0, sem1)
    outs = (outv0, outv1)

    def fire(c):
        buf = bufs[c % 2]
        sem = sems[c % 2]
        for j in range(NG):
            pltpu.async_copy(
                w_hbm.at[idx_v.at[c * NG + j]],
                buf.at[pl.ds(j * GN, GN)],
                sem,
            )

    fire(0)
    store_handles = [None, None]
    for c in range(NCHUNK):
        if c + 1 < NCHUNK:
            fire(c + 1)
        buf = bufs[c % 2]
        pltpu.make_async_copy(w_hbm.at[pl.ds(0, TPC)], buf,
                              sems[c % 2]).wait()
        out_v = outs[c % 2]
        if store_handles[c % 2] is not None:
            store_handles[c % 2].wait()

        def accum(b, _, buf=buf, out_v=out_v):
            cols = [pl.ds(eb * LANES, LANES) for eb in range(E // LANES)]
            accs = [buf[b * S, col] for col in cols]
            for s in range(1, S):
                row = b * S + s
                accs = [acc + buf[row, col] for acc, col in zip(accs, cols)]
            for col, acc in zip(cols, accs):
                out_v[b, col] = acc
            return 0

        lax.fori_loop(0, CB, accum, 0)
        store_handles[c % 2] = pltpu.async_copy(
            out_v, out_hbm.at[pl.ds(wid * BPW + c * CB, CB)], sem_out)
    for h in store_handles:
        h.wait()


def kernel(utterance, W):
    utt = utterance.astype(jnp.int32).reshape(NW, IDX_ROWS, GN)
    return _sc_embed_sum(utt, W)
